# Initial kernel scaffold; baseline (speedup 1.0000x reference)
import jax
import jax.numpy as jnp
from jax.experimental import pallas as pl


def kernel(x, edge_index, edge_attr, latent_vector, Wm, bm, Wih, Whh, bih, bhh, gWm, gbm, gWih, gWhh, gbih, gbhh, Wg, bg, Wp, bp, fc1_W, fc1_b, fc2_W, fc2_b):
    raise NotImplementedError("write your pallas kernel here")


# speedup vs baseline: 6.0839x; 6.0839x over previous
import functools

import jax
import jax.numpy as jnp
from jax import lax
from jax.experimental import pallas as pl
from jax.experimental.pallas import tpu as pltpu
from jax.experimental.pallas import tpu_sc as plsc

N = 10000
E = 320000
D = 128
DE = 16
NF = 64

NC = 2
NS = 16
NW = NC * NS
EPW = E // NW
CH = 125
NCH = EPW // CH
NPAD = 10112
RPT = NPAD // NS
EW = 32

def _sc_spmm_body(src_hbm, dst_hbm, hm_hbm, zer_hbm, out_hbm,
                  src_v, dst_v, gbuf, sem, agg_sh):
    c = lax.axis_index("c")
    s = lax.axis_index("s")
    wid = s * NC + c
    pltpu.sync_copy(zer_hbm, agg_sh.at[pl.ds(s * RPT, RPT)])
    pltpu.sync_copy(src_hbm.at[wid], src_v)
    pltpu.sync_copy(dst_hbm.at[wid], dst_v)
    plsc.subcore_barrier()

    def chunk(j, carry):
        pltpu.async_copy(hm_hbm.at[src_v.at[j]], gbuf, sem).wait()
        pltpu.sync_copy(gbuf, agg_sh.at[dst_v.at[j]], add=True)
        return carry

    lax.fori_loop(0, NCH, chunk, 0, unroll=False)
    plsc.subcore_barrier()
    pltpu.sync_copy(agg_sh.at[pl.ds(s * RPT, RPT)],
                    out_hbm.at[c].at[pl.ds(s * RPT, RPT)])


def _sc_eagg_body(dst_hbm, ea_hbm, zer_hbm, out_hbm,
                  dst_v, pbuf, agg_sh):
    c = lax.axis_index("c")
    s = lax.axis_index("s")
    wid = s * NC + c
    pltpu.sync_copy(zer_hbm, agg_sh.at[pl.ds(s * RPT, RPT)])
    pltpu.sync_copy(dst_hbm.at[wid], dst_v)
    plsc.subcore_barrier()

    def chunk(j, carry):
        pltpu.sync_copy(ea_hbm.at[wid].at[j], pbuf)
        pltpu.sync_copy(pbuf, agg_sh.at[dst_v.at[j]], add=True)
        return carry

    lax.fori_loop(0, NCH, chunk, 0, unroll=False)
    plsc.subcore_barrier()
    pltpu.sync_copy(agg_sh.at[pl.ds(s * RPT, RPT)],
                    out_hbm.at[c].at[pl.ds(s * RPT, RPT)])


@functools.lru_cache(maxsize=None)
def _sc_kernels():
    mesh = plsc.VectorSubcoreMesh(core_axis_name="c", subcore_axis_name="s")
    spmm = pl.kernel(
        _sc_spmm_body,
        out_type=jax.ShapeDtypeStruct((NC, NPAD, D), jnp.float32),
        mesh=mesh,
        scratch_types=[
            pltpu.VMEM((NCH, CH), jnp.int32),
            pltpu.VMEM((NCH, CH), jnp.int32),
            pltpu.VMEM((CH, D), jnp.float32),
            pltpu.SemaphoreType.DMA,
            pltpu.VMEM_SHARED((NPAD, D), jnp.float32),
        ],
    )
    eagg = pl.kernel(
        _sc_eagg_body,
        out_type=jax.ShapeDtypeStruct((NC, NPAD, EW), jnp.float32),
        mesh=mesh,
        scratch_types=[
            pltpu.VMEM((NCH, CH), jnp.int32),
            pltpu.VMEM((CH, EW), jnp.float32),
            pltpu.VMEM_SHARED((NPAD, EW), jnp.float32),
        ],
    )
    return spmm, eagg


def _sc_spmm(src_r, dst_r, hm, zer_d):
    return _sc_kernels()[0](src_r, dst_r, hm, zer_d)


def _sc_eagg(dst_r, ea_ext, zer_e):
    return _sc_kernels()[1](dst_r, ea_ext, zer_e)


RB = 1000
NB = N // RB


def _mm_body(x_ref, w_ref, o_ref):
    o_ref[...] = jnp.dot(x_ref[...], w_ref[...],
                         preferred_element_type=jnp.float32)


def _tc_matmul(x, w):
    m, k = x.shape
    _, n = w.shape
    return pl.pallas_call(
        _mm_body,
        grid=(m // RB,),
        in_specs=[
            pl.BlockSpec((RB, k), lambda i: (i, 0)),
            pl.BlockSpec((k, n), lambda i: (0, 0)),
        ],
        out_specs=pl.BlockSpec((RB, n), lambda i: (i, 0)),
        out_shape=jax.ShapeDtypeStruct((m, n), jnp.float32),
    )(x, w)


def _gru_body(with_next, h_ref, spm_ref, eagg_ref, wext_ref, wih_ref, whh_ref,
              bih_ref, bhh_ref, wnext_ref, hout_ref, hmout_ref):
    h = h_ref[...]
    agg = (spm_ref[0] + spm_ref[1]
           + jnp.dot(eagg_ref[0] + eagg_ref[1], wext_ref[...],
                     preferred_element_type=jnp.float32))
    gi = jnp.dot(agg, wih_ref[...], preferred_element_type=jnp.float32) \
        + bih_ref[...]
    gh = jnp.dot(h, whh_ref[...], preferred_element_type=jnp.float32) \
        + bhh_ref[...]
    r = jax.nn.sigmoid(gi[:, :D] + gh[:, :D])
    z = jax.nn.sigmoid(gi[:, D:2 * D] + gh[:, D:2 * D])
    n = jnp.tanh(gi[:, 2 * D:] + r * gh[:, 2 * D:])
    hn = (1.0 - z) * n + z * h
    hout_ref[...] = hn
    if with_next:
        hmout_ref[...] = jnp.dot(hn, wnext_ref[...],
                                 preferred_element_type=jnp.float32)


def _tc_gru(h, spm, eagg, wext, wih, whh, bih, bhh, wnext):
    with_next = wnext is not None
    if not with_next:
        wnext = jnp.zeros((D, D), jnp.float32)
    out_shapes = [jax.ShapeDtypeStruct((N, D), jnp.float32),
                  jax.ShapeDtypeStruct((N, D), jnp.float32)]
    outs = pl.pallas_call(
        functools.partial(_gru_body, with_next),
        grid=(NB,),
        in_specs=[
            pl.BlockSpec((RB, D), lambda i: (i, 0)),
            pl.BlockSpec((NC, RB, D), lambda i: (0, i, 0)),
            pl.BlockSpec((NC, RB, EW), lambda i: (0, i, 0)),
            pl.BlockSpec((EW, D), lambda i: (0, 0)),
            pl.BlockSpec((D, 3 * D), lambda i: (0, 0)),
            pl.BlockSpec((D, 3 * D), lambda i: (0, 0)),
            pl.BlockSpec((1, 3 * D), lambda i: (0, 0)),
            pl.BlockSpec((1, 3 * D), lambda i: (0, 0)),
            pl.BlockSpec((D, D), lambda i: (0, 0)),
        ],
        out_specs=[pl.BlockSpec((RB, D), lambda i: (i, 0)),
                   pl.BlockSpec((RB, D), lambda i: (i, 0))],
        out_shape=out_shapes,
    )(h, spm, eagg, wext, wih, whh, bih, bhh, wnext)
    return outs


def _readout_body(h_ref, wg_ref, bg_ref, wp_ref, bp_ref, lat_ref,
                  fc1_ref, fc1b_ref, fc2_ref, fc2b_ref, out_ref, acc_ref):
    i = pl.program_id(0)
    h = h_ref[...]
    gate = jax.nn.sigmoid(
        jnp.dot(h, wg_ref[...], preferred_element_type=jnp.float32)
        + bg_ref[...])
    proj = jnp.tanh(
        jnp.dot(h, wp_ref[...], preferred_element_type=jnp.float32)
        + bp_ref[...])
    part = jnp.sum(gate * proj, axis=0, keepdims=True)

    @pl.when(i == 0)
    def _():
        acc_ref[...] = part

    @pl.when(i > 0)
    def _():
        acc_ref[...] += part

    @pl.when(i == pl.num_programs(0) - 1)
    def _():
        cat = jnp.concatenate([acc_ref[...], lat_ref[...]], axis=1)
        hid = jax.nn.relu(
            jnp.dot(cat, fc1_ref[...], preferred_element_type=jnp.float32)
            + fc1b_ref[...])
        logits = jnp.dot(hid, fc2_ref[...],
                         preferred_element_type=jnp.float32) + fc2b_ref[...]
        out_ref[...] = jax.nn.softmax(logits, axis=-1)


def _tc_readout(h, wg, bg, wp, bp, lat, fc1w, fc1b, fc2w, fc2b):
    return pl.pallas_call(
        _readout_body,
        grid=(NB,),
        in_specs=[
            pl.BlockSpec((RB, D), lambda i: (i, 0)),
            pl.BlockSpec((D, D), lambda i: (0, 0)),
            pl.BlockSpec((1, D), lambda i: (0, 0)),
            pl.BlockSpec((D, D), lambda i: (0, 0)),
            pl.BlockSpec((1, D), lambda i: (0, 0)),
            pl.BlockSpec((1, D), lambda i: (0, 0)),
            pl.BlockSpec((2 * D, D), lambda i: (0, 0)),
            pl.BlockSpec((1, D), lambda i: (0, 0)),
            pl.BlockSpec((D, NF), lambda i: (0, 0)),
            pl.BlockSpec((1, NF), lambda i: (0, 0)),
        ],
        out_specs=pl.BlockSpec((1, NF), lambda i: (0, 0)),
        out_shape=jax.ShapeDtypeStruct((1, NF), jnp.float32),
        scratch_shapes=[pltpu.VMEM((1, D), jnp.float32)],
    )(h, wg, bg, wp, bp, lat, fc1w, fc1b, fc2w, fc2b)


def kernel(x, edge_index, edge_attr, latent_vector, Wm, bm, Wih, Whh, bih,
           bhh, gWm, gbm, gWih, gWhh, gbih, gbhh, Wg, bg, Wp, bp, fc1_W,
           fc1_b, fc2_W, fc2_b):
    src_r = edge_index[0].reshape(NW, NCH, CH)
    dst_r = edge_index[1].reshape(NW, NCH, CH)
    ea_ext = jnp.concatenate(
        [edge_attr,
         jnp.ones((E, 1), jnp.float32),
         jnp.zeros((E, EW - DE - 1), jnp.float32)], axis=1
    ).reshape(NW, NCH, CH, EW)
    zer_d = jnp.zeros((RPT, D), jnp.float32)
    zer_e = jnp.zeros((RPT, EW), jnp.float32)

    WmA = [Wm[0, :D], Wm[1, :D], gWm[0, :D], gWm[1, :D]]
    pad = jnp.zeros((EW - DE - 1, D), jnp.float32)
    Wext = [jnp.concatenate([Wm[0, D:], bm[0][None], pad], 0),
            jnp.concatenate([Wm[1, D:], bm[1][None], pad], 0),
            jnp.concatenate([gWm[0, D:], gbm[0][None], pad], 0),
            jnp.concatenate([gWm[1, D:], gbm[1][None], pad], 0)]
    WihL = [Wih[0], Wih[1], gWih[0], gWih[1]]
    WhhL = [Whh[0], Whh[1], gWhh[0], gWhh[1]]
    bihL = [bih[0][None], bih[1][None], gbih[0][None], gbih[1][None]]
    bhhL = [bhh[0][None], bhh[1][None], gbhh[0][None], gbhh[1][None]]

    eagg = _sc_eagg(dst_r, ea_ext, zer_e)

    h = x
    hm = _tc_matmul(x, WmA[0])
    for l in range(4):
        spm = _sc_spmm(src_r, dst_r, hm, zer_d)
        wnext = WmA[l + 1] if l < 3 else None
        h, hm = _tc_gru(h, spm, eagg, Wext[l], WihL[l], WhhL[l],
                        bihL[l], bhhL[l], wnext)

    return _tc_readout(h, Wg, bg[None], Wp, bp[None], latent_vector,
                       fc1_W, fc1_b[None], fc2_W, fc2_b[None])


# speedup vs baseline: 7.2910x; 1.1984x over previous
import functools

import jax
import jax.numpy as jnp
from jax import lax
from jax.experimental import pallas as pl
from jax.experimental.pallas import tpu as pltpu
from jax.experimental.pallas import tpu_sc as plsc

N = 10000
E = 320000
D = 128
DE = 16
NF = 64

NC = 2
NS = 16
NW = NC * NS
EPW = E // NW
CH = 125
NCH = EPW // CH
NPAD = 10112
RPT = NPAD // NS
EW = 32

def _sc_spmm_body(src_hbm, dst_hbm, hm_hbm, zer_hbm, out_hbm,
                  dst_v, sp0, sp1, gbuf0, gbuf1,
                  gsem0, gsem1, isem0, isem1, agg_sh):
    c = lax.axis_index("c")
    s = lax.axis_index("s")
    wid = s * NC + c
    pltpu.sync_copy(zer_hbm, agg_sh.at[pl.ds(s * RPT, RPT)])
    pltpu.sync_copy(dst_hbm.at[wid], dst_v)
    pltpu.sync_copy(src_hbm.at[wid, pl.ds(0, 2)], sp0)
    pltpu.sync_copy(src_hbm.at[wid, pl.ds(2, 2)], sp1)
    plsc.subcore_barrier()

    pltpu.async_copy(hm_hbm.at[sp0.at[0]], gbuf0, gsem0)
    ni = NCH // 4

    def quad(i, carry):
        c0 = 4 * i
        pltpu.async_copy(hm_hbm.at[sp0.at[1]], gbuf1, gsem1)
        pltpu.make_async_copy(hm_hbm.at[sp0.at[0]], gbuf0, gsem0).wait()
        pltpu.sync_copy(gbuf0, agg_sh.at[dst_v.at[c0]], add=True)

        @pl.when(i > 0)
        def _():
            pltpu.make_async_copy(src_hbm.at[wid, pl.ds(0, 2)], sp1,
                                  isem1).wait()

        pltpu.async_copy(hm_hbm.at[sp1.at[0]], gbuf0, gsem0)
        pltpu.make_async_copy(hm_hbm.at[sp0.at[1]], gbuf1, gsem1).wait()
        pltpu.sync_copy(gbuf1, agg_sh.at[dst_v.at[c0 + 1]], add=True)

        @pl.when(i < ni - 1)
        def _():
            pltpu.async_copy(src_hbm.at[wid, pl.ds(c0 + 4, 2)], sp0, isem0)

        pltpu.async_copy(hm_hbm.at[sp1.at[1]], gbuf1, gsem1)
        pltpu.make_async_copy(hm_hbm.at[sp1.at[0]], gbuf0, gsem0).wait()
        pltpu.sync_copy(gbuf0, agg_sh.at[dst_v.at[c0 + 2]], add=True)
        pltpu.make_async_copy(hm_hbm.at[sp1.at[1]], gbuf1, gsem1).wait()
        pltpu.sync_copy(gbuf1, agg_sh.at[dst_v.at[c0 + 3]], add=True)

        @pl.when(i < ni - 1)
        def _():
            pltpu.async_copy(src_hbm.at[wid, pl.ds(c0 + 6, 2)], sp1, isem1)
            pltpu.make_async_copy(src_hbm.at[wid, pl.ds(0, 2)], sp0,
                                  isem0).wait()
            pltpu.async_copy(hm_hbm.at[sp0.at[0]], gbuf0, gsem0)

        return carry

    lax.fori_loop(0, ni, quad, 0, unroll=False)
    plsc.subcore_barrier()
    pltpu.sync_copy(agg_sh.at[pl.ds(s * RPT, RPT)],
                    out_hbm.at[c].at[pl.ds(s * RPT, RPT)])


def _sc_eagg_body(dst_hbm, ea_hbm, zer_hbm, out_hbm,
                  dst_v, pbuf0, pbuf1, sem0, sem1, agg_sh):
    c = lax.axis_index("c")
    s = lax.axis_index("s")
    wid = s * NC + c
    pltpu.sync_copy(zer_hbm, agg_sh.at[pl.ds(s * RPT, RPT)])
    pltpu.sync_copy(dst_hbm.at[wid], dst_v)
    plsc.subcore_barrier()

    pltpu.async_copy(ea_hbm.at[wid].at[0], pbuf0, sem0)
    npair = NCH // 2

    def pair(i, carry):
        a = 2 * i
        pltpu.async_copy(ea_hbm.at[wid].at[a + 1], pbuf1, sem1)
        pltpu.make_async_copy(ea_hbm.at[wid].at[a], pbuf0, sem0).wait()
        pltpu.sync_copy(pbuf0, agg_sh.at[dst_v.at[a]], add=True)

        @pl.when(i < npair - 1)
        def _():
            pltpu.async_copy(ea_hbm.at[wid].at[a + 2], pbuf0, sem0)

        pltpu.make_async_copy(ea_hbm.at[wid].at[a + 1], pbuf1, sem1).wait()
        pltpu.sync_copy(pbuf1, agg_sh.at[dst_v.at[a + 1]], add=True)
        return carry

    lax.fori_loop(0, npair, pair, 0, unroll=False)
    plsc.subcore_barrier()
    pltpu.sync_copy(agg_sh.at[pl.ds(s * RPT, RPT)],
                    out_hbm.at[c].at[pl.ds(s * RPT, RPT)])


@functools.lru_cache(maxsize=None)
def _sc_kernels():
    mesh = plsc.VectorSubcoreMesh(core_axis_name="c", subcore_axis_name="s")
    spmm = pl.kernel(
        _sc_spmm_body,
        out_type=jax.ShapeDtypeStruct((NC, NPAD, D), jnp.float32),
        mesh=mesh,
        scratch_types=[
            pltpu.VMEM((NCH, CH), jnp.int32),
            pltpu.VMEM((2, CH), jnp.int32),
            pltpu.VMEM((2, CH), jnp.int32),
            pltpu.VMEM((CH, D), jnp.float32),
            pltpu.VMEM((CH, D), jnp.float32),
            pltpu.SemaphoreType.DMA,
            pltpu.SemaphoreType.DMA,
            pltpu.SemaphoreType.DMA,
            pltpu.SemaphoreType.DMA,
            pltpu.VMEM_SHARED((NPAD, D), jnp.float32),
        ],
    )
    eagg = pl.kernel(
        _sc_eagg_body,
        out_type=jax.ShapeDtypeStruct((NC, NPAD, EW), jnp.float32),
        mesh=mesh,
        scratch_types=[
            pltpu.VMEM((NCH, CH), jnp.int32),
            pltpu.VMEM((CH, EW), jnp.float32),
            pltpu.VMEM((CH, EW), jnp.float32),
            pltpu.SemaphoreType.DMA,
            pltpu.SemaphoreType.DMA,
            pltpu.VMEM_SHARED((NPAD, EW), jnp.float32),
        ],
    )
    return spmm, eagg


def _sc_spmm(src_r, dst_r, hm, zer_d):
    return _sc_kernels()[0](src_r, dst_r, hm, zer_d)


def _sc_eagg(dst_r, ea_ext, zer_e):
    return _sc_kernels()[1](dst_r, ea_ext, zer_e)


RB = 1000
NB = N // RB


def _mm_body(x_ref, w_ref, o_ref):
    o_ref[...] = jnp.dot(x_ref[...], w_ref[...],
                         preferred_element_type=jnp.float32)


def _tc_matmul(x, w):
    m, k = x.shape
    _, n = w.shape
    return pl.pallas_call(
        _mm_body,
        grid=(m // RB,),
        in_specs=[
            pl.BlockSpec((RB, k), lambda i: (i, 0)),
            pl.BlockSpec((k, n), lambda i: (0, 0)),
        ],
        out_specs=pl.BlockSpec((RB, n), lambda i: (i, 0)),
        out_shape=jax.ShapeDtypeStruct((m, n), jnp.float32),
    )(x, w)


def _gru_body(with_next, h_ref, spm_ref, eagg_ref, wext_ref, wih_ref, whh_ref,
              bih_ref, bhh_ref, wnext_ref, hout_ref, hmout_ref):
    h = h_ref[...]
    agg = (spm_ref[0] + spm_ref[1]
           + jnp.dot(eagg_ref[0] + eagg_ref[1], wext_ref[...],
                     preferred_element_type=jnp.float32))
    gi = jnp.dot(agg, wih_ref[...], preferred_element_type=jnp.float32) \
        + bih_ref[...]
    gh = jnp.dot(h, whh_ref[...], preferred_element_type=jnp.float32) \
        + bhh_ref[...]
    r = jax.nn.sigmoid(gi[:, :D] + gh[:, :D])
    z = jax.nn.sigmoid(gi[:, D:2 * D] + gh[:, D:2 * D])
    n = jnp.tanh(gi[:, 2 * D:] + r * gh[:, 2 * D:])
    hn = (1.0 - z) * n + z * h
    hout_ref[...] = hn
    if with_next:
        hmout_ref[...] = jnp.dot(hn, wnext_ref[...],
                                 preferred_element_type=jnp.float32)


def _tc_gru(h, spm, eagg, wext, wih, whh, bih, bhh, wnext):
    with_next = wnext is not None
    if not with_next:
        wnext = jnp.zeros((D, D), jnp.float32)
    out_shapes = [jax.ShapeDtypeStruct((N, D), jnp.float32),
                  jax.ShapeDtypeStruct((N, D), jnp.float32)]
    outs = pl.pallas_call(
        functools.partial(_gru_body, with_next),
        grid=(NB,),
        in_specs=[
            pl.BlockSpec((RB, D), lambda i: (i, 0)),
            pl.BlockSpec((NC, RB, D), lambda i: (0, i, 0)),
            pl.BlockSpec((NC, RB, EW), lambda i: (0, i, 0)),
            pl.BlockSpec((EW, D), lambda i: (0, 0)),
            pl.BlockSpec((D, 3 * D), lambda i: (0, 0)),
            pl.BlockSpec((D, 3 * D), lambda i: (0, 0)),
            pl.BlockSpec((1, 3 * D), lambda i: (0, 0)),
            pl.BlockSpec((1, 3 * D), lambda i: (0, 0)),
            pl.BlockSpec((D, D), lambda i: (0, 0)),
        ],
        out_specs=[pl.BlockSpec((RB, D), lambda i: (i, 0)),
                   pl.BlockSpec((RB, D), lambda i: (i, 0))],
        out_shape=out_shapes,
    )(h, spm, eagg, wext, wih, whh, bih, bhh, wnext)
    return outs


def _readout_body(h_ref, wg_ref, bg_ref, wp_ref, bp_ref, lat_ref,
                  fc1_ref, fc1b_ref, fc2_ref, fc2b_ref, out_ref, acc_ref):
    i = pl.program_id(0)
    h = h_ref[...]
    gate = jax.nn.sigmoid(
        jnp.dot(h, wg_ref[...], preferred_element_type=jnp.float32)
        + bg_ref[...])
    proj = jnp.tanh(
        jnp.dot(h, wp_ref[...], preferred_element_type=jnp.float32)
        + bp_ref[...])
    part = jnp.sum(gate * proj, axis=0, keepdims=True)

    @pl.when(i == 0)
    def _():
        acc_ref[...] = part

    @pl.when(i > 0)
    def _():
        acc_ref[...] += part

    @pl.when(i == pl.num_programs(0) - 1)
    def _():
        cat = jnp.concatenate([acc_ref[...], lat_ref[...]], axis=1)
        hid = jax.nn.relu(
            jnp.dot(cat, fc1_ref[...], preferred_element_type=jnp.float32)
            + fc1b_ref[...])
        logits = jnp.dot(hid, fc2_ref[...],
                         preferred_element_type=jnp.float32) + fc2b_ref[...]
        out_ref[...] = jax.nn.softmax(logits, axis=-1)


def _tc_readout(h, wg, bg, wp, bp, lat, fc1w, fc1b, fc2w, fc2b):
    return pl.pallas_call(
        _readout_body,
        grid=(NB,),
        in_specs=[
            pl.BlockSpec((RB, D), lambda i: (i, 0)),
            pl.BlockSpec((D, D), lambda i: (0, 0)),
            pl.BlockSpec((1, D), lambda i: (0, 0)),
            pl.BlockSpec((D, D), lambda i: (0, 0)),
            pl.BlockSpec((1, D), lambda i: (0, 0)),
            pl.BlockSpec((1, D), lambda i: (0, 0)),
            pl.BlockSpec((2 * D, D), lambda i: (0, 0)),
            pl.BlockSpec((1, D), lambda i: (0, 0)),
            pl.BlockSpec((D, NF), lambda i: (0, 0)),
            pl.BlockSpec((1, NF), lambda i: (0, 0)),
        ],
        out_specs=pl.BlockSpec((1, NF), lambda i: (0, 0)),
        out_shape=jax.ShapeDtypeStruct((1, NF), jnp.float32),
        scratch_shapes=[pltpu.VMEM((1, D), jnp.float32)],
    )(h, wg, bg, wp, bp, lat, fc1w, fc1b, fc2w, fc2b)


def kernel(x, edge_index, edge_attr, latent_vector, Wm, bm, Wih, Whh, bih,
           bhh, gWm, gbm, gWih, gWhh, gbih, gbhh, Wg, bg, Wp, bp, fc1_W,
           fc1_b, fc2_W, fc2_b):
    src_r = edge_index[0].reshape(NW, NCH, CH)
    dst_r = edge_index[1].reshape(NW, NCH, CH)
    ea_ext = jnp.concatenate(
        [edge_attr,
         jnp.ones((E, 1), jnp.float32),
         jnp.zeros((E, EW - DE - 1), jnp.float32)], axis=1
    ).reshape(NW, NCH, CH, EW)
    zer_d = jnp.zeros((RPT, D), jnp.float32)
    zer_e = jnp.zeros((RPT, EW), jnp.float32)

    WmA = [Wm[0, :D], Wm[1, :D], gWm[0, :D], gWm[1, :D]]
    pad = jnp.zeros((EW - DE - 1, D), jnp.float32)
    Wext = [jnp.concatenate([Wm[0, D:], bm[0][None], pad], 0),
            jnp.concatenate([Wm[1, D:], bm[1][None], pad], 0),
            jnp.concatenate([gWm[0, D:], gbm[0][None], pad], 0),
            jnp.concatenate([gWm[1, D:], gbm[1][None], pad], 0)]
    WihL = [Wih[0], Wih[1], gWih[0], gWih[1]]
    WhhL = [Whh[0], Whh[1], gWhh[0], gWhh[1]]
    bihL = [bih[0][None], bih[1][None], gbih[0][None], gbih[1][None]]
    bhhL = [bhh[0][None], bhh[1][None], gbhh[0][None], gbhh[1][None]]

    eagg = _sc_eagg(dst_r, ea_ext, zer_e)

    h = x
    hm = _tc_matmul(x, WmA[0])
    for l in range(4):
        spm = _sc_spmm(src_r, dst_r, hm, zer_d)
        wnext = WmA[l + 1] if l < 3 else None
        h, hm = _tc_gru(h, spm, eagg, Wext[l], WihL[l], WhhL[l],
                        bihL[l], bhhL[l], wnext)

    return _tc_readout(h, Wg, bg[None], Wp, bp[None], latent_vector,
                       fc1_W, fc1_b[None], fc2_W, fc2_b[None])


# speedup vs baseline: 7.3124x; 1.0029x over previous
import functools

import jax
import jax.numpy as jnp
from jax import lax
from jax.experimental import pallas as pl
from jax.experimental.pallas import tpu as pltpu
from jax.experimental.pallas import tpu_sc as plsc

N = 10000
E = 320000
D = 128
DE = 16
NF = 64

NC = 2
NS = 16
NW = NC * NS
EPW = E // NW
CH = 125
NCH = EPW // CH
NPAD = 10112
RPT = NPAD // NS
EW = 32

def _sc_spmm_body(src_hbm, dst_hbm, hm_hbm, zer_hbm, out_hbm,
                  src_v, dst_v, gb0, gb1,
                  is0, is1, is2, is3, gs0, gs1, as0, as1, agg_sh):
    c = lax.axis_index("c")
    s = lax.axis_index("s")
    wid = s * NC + c
    pltpu.sync_copy(zer_hbm, agg_sh.at[pl.ds(s * RPT, RPT)])
    pltpu.sync_copy(dst_hbm.at[wid], dst_v)
    pltpu.sync_copy(src_hbm.at[wid, pl.ds(0, 3)], src_v.at[pl.ds(0, 3)])
    plsc.subcore_barrier()

    gbufs = [gb0, gb1]
    gsems = [gs0, gs1]
    asems = [as0, as1]
    isems = [is0, is1, is2, is3]

    pltpu.async_copy(hm_hbm.at[src_v.at[0]], gbufs[0], gsems[0])

    def rnd(r, carry):
        i0 = r * 4
        for b in range(4):
            ch = i0 + b
            b2 = b % 2
            pltpu.make_async_copy(hm_hbm.at[src_v.at[b]], gbufs[b2],
                                  gsems[b2]).wait()
            pltpu.async_copy(gbufs[b2], agg_sh.at[dst_v.at[ch]],
                             asems[b2], add=True)

            @pl.when(ch + 3 < NCH)
            def _():
                pltpu.async_copy(src_hbm.at[wid].at[ch + 3],
                                 src_v.at[(b + 3) % 4], isems[(b + 3) % 4])

            @pl.when(ch + 1 < NCH)
            def _():
                @pl.when(ch >= 1)
                def _():
                    pltpu.make_async_copy(
                        gbufs[1 - b2], agg_sh.at[dst_v.at[0]],
                        asems[1 - b2]).wait()

                @pl.when(ch + 1 >= 3)
                def _():
                    pltpu.make_async_copy(
                        src_hbm.at[wid].at[0], src_v.at[(b + 1) % 4],
                        isems[(b + 1) % 4]).wait()
                pltpu.async_copy(hm_hbm.at[src_v.at[(b + 1) % 4]],
                                 gbufs[1 - b2], gsems[1 - b2])
        return carry

    lax.fori_loop(0, NCH // 4, rnd, 0, unroll=False)
    for b2 in range(2):
        pltpu.make_async_copy(gbufs[b2], agg_sh.at[dst_v.at[0]],
                              asems[b2]).wait()
    plsc.subcore_barrier()
    pltpu.sync_copy(agg_sh.at[pl.ds(s * RPT, RPT)],
                    out_hbm.at[c].at[pl.ds(s * RPT, RPT)])


def _sc_eagg_body(dst_hbm, ea_hbm, zer_hbm, out_hbm,
                  dst_v, pbuf0, pbuf1, sem0, sem1, agg_sh):
    c = lax.axis_index("c")
    s = lax.axis_index("s")
    wid = s * NC + c
    pltpu.sync_copy(zer_hbm, agg_sh.at[pl.ds(s * RPT, RPT)])
    pltpu.sync_copy(dst_hbm.at[wid], dst_v)
    plsc.subcore_barrier()

    pltpu.async_copy(ea_hbm.at[wid].at[0], pbuf0, sem0)
    npair = NCH // 2

    def pair(i, carry):
        a = 2 * i
        pltpu.async_copy(ea_hbm.at[wid].at[a + 1], pbuf1, sem1)
        pltpu.make_async_copy(ea_hbm.at[wid].at[a], pbuf0, sem0).wait()
        pltpu.sync_copy(pbuf0, agg_sh.at[dst_v.at[a]], add=True)

        @pl.when(i < npair - 1)
        def _():
            pltpu.async_copy(ea_hbm.at[wid].at[a + 2], pbuf0, sem0)

        pltpu.make_async_copy(ea_hbm.at[wid].at[a + 1], pbuf1, sem1).wait()
        pltpu.sync_copy(pbuf1, agg_sh.at[dst_v.at[a + 1]], add=True)
        return carry

    lax.fori_loop(0, npair, pair, 0, unroll=False)
    plsc.subcore_barrier()
    pltpu.sync_copy(agg_sh.at[pl.ds(s * RPT, RPT)],
                    out_hbm.at[c].at[pl.ds(s * RPT, RPT)])


@functools.lru_cache(maxsize=None)
def _sc_kernels():
    mesh = plsc.VectorSubcoreMesh(core_axis_name="c", subcore_axis_name="s")
    spmm = pl.kernel(
        _sc_spmm_body,
        out_type=jax.ShapeDtypeStruct((NC, NPAD, D), jnp.float32),
        mesh=mesh,
        scratch_types=[
            pltpu.VMEM((4, CH), jnp.int32),
            pltpu.VMEM((NCH, CH), jnp.int32),
            pltpu.VMEM((CH, D), jnp.float32),
            pltpu.VMEM((CH, D), jnp.float32),
            pltpu.SemaphoreType.DMA,
            pltpu.SemaphoreType.DMA,
            pltpu.SemaphoreType.DMA,
            pltpu.SemaphoreType.DMA,
            pltpu.SemaphoreType.DMA,
            pltpu.SemaphoreType.DMA,
            pltpu.SemaphoreType.DMA,
            pltpu.SemaphoreType.DMA,
            pltpu.VMEM_SHARED((NPAD, D), jnp.float32),
        ],
    )
    eagg = pl.kernel(
        _sc_eagg_body,
        out_type=jax.ShapeDtypeStruct((NC, NPAD, EW), jnp.float32),
        mesh=mesh,
        scratch_types=[
            pltpu.VMEM((NCH, CH), jnp.int32),
            pltpu.VMEM((CH, EW), jnp.float32),
            pltpu.VMEM((CH, EW), jnp.float32),
            pltpu.SemaphoreType.DMA,
            pltpu.SemaphoreType.DMA,
            pltpu.VMEM_SHARED((NPAD, EW), jnp.float32),
        ],
    )
    return spmm, eagg


def _sc_spmm(src_r, dst_r, hm, zer_d):
    return _sc_kernels()[0](src_r, dst_r, hm, zer_d)


def _sc_eagg(dst_r, ea_ext, zer_e):
    return _sc_kernels()[1](dst_r, ea_ext, zer_e)


RB = 1000
NB = N // RB


def _mm_body(x_ref, w_ref, o_ref):
    o_ref[...] = jnp.dot(x_ref[...], w_ref[...],
                         preferred_element_type=jnp.float32)


def _tc_matmul(x, w):
    m, k = x.shape
    _, n = w.shape
    return pl.pallas_call(
        _mm_body,
        grid=(m // RB,),
        in_specs=[
            pl.BlockSpec((RB, k), lambda i: (i, 0)),
            pl.BlockSpec((k, n), lambda i: (0, 0)),
        ],
        out_specs=pl.BlockSpec((RB, n), lambda i: (i, 0)),
        out_shape=jax.ShapeDtypeStruct((m, n), jnp.float32),
    )(x, w)


def _gru_body(with_next, h_ref, spm_ref, eagg_ref, wext_ref, wih_ref, whh_ref,
              bih_ref, bhh_ref, wnext_ref, hout_ref, hmout_ref):
    h = h_ref[...]
    agg = (spm_ref[0] + spm_ref[1]
           + jnp.dot(eagg_ref[0] + eagg_ref[1], wext_ref[...],
                     preferred_element_type=jnp.float32))
    gi = jnp.dot(agg, wih_ref[...], preferred_element_type=jnp.float32) \
        + bih_ref[...]
    gh = jnp.dot(h, whh_ref[...], preferred_element_type=jnp.float32) \
        + bhh_ref[...]
    r = jax.nn.sigmoid(gi[:, :D] + gh[:, :D])
    z = jax.nn.sigmoid(gi[:, D:2 * D] + gh[:, D:2 * D])
    n = jnp.tanh(gi[:, 2 * D:] + r * gh[:, 2 * D:])
    hn = (1.0 - z) * n + z * h
    hout_ref[...] = hn
    if with_next:
        hmout_ref[...] = jnp.dot(hn, wnext_ref[...],
                                 preferred_element_type=jnp.float32)


def _tc_gru(h, spm, eagg, wext, wih, whh, bih, bhh, wnext):
    with_next = wnext is not None
    if not with_next:
        wnext = jnp.zeros((D, D), jnp.float32)
    out_shapes = [jax.ShapeDtypeStruct((N, D), jnp.float32),
                  jax.ShapeDtypeStruct((N, D), jnp.float32)]
    outs = pl.pallas_call(
        functools.partial(_gru_body, with_next),
        grid=(NB,),
        in_specs=[
            pl.BlockSpec((RB, D), lambda i: (i, 0)),
            pl.BlockSpec((NC, RB, D), lambda i: (0, i, 0)),
            pl.BlockSpec((NC, RB, EW), lambda i: (0, i, 0)),
            pl.BlockSpec((EW, D), lambda i: (0, 0)),
            pl.BlockSpec((D, 3 * D), lambda i: (0, 0)),
            pl.BlockSpec((D, 3 * D), lambda i: (0, 0)),
            pl.BlockSpec((1, 3 * D), lambda i: (0, 0)),
            pl.BlockSpec((1, 3 * D), lambda i: (0, 0)),
            pl.BlockSpec((D, D), lambda i: (0, 0)),
        ],
        out_specs=[pl.BlockSpec((RB, D), lambda i: (i, 0)),
                   pl.BlockSpec((RB, D), lambda i: (i, 0))],
        out_shape=out_shapes,
    )(h, spm, eagg, wext, wih, whh, bih, bhh, wnext)
    return outs


def _readout_body(h_ref, wg_ref, bg_ref, wp_ref, bp_ref, lat_ref,
                  fc1_ref, fc1b_ref, fc2_ref, fc2b_ref, out_ref, acc_ref):
    i = pl.program_id(0)
    h = h_ref[...]
    gate = jax.nn.sigmoid(
        jnp.dot(h, wg_ref[...], preferred_element_type=jnp.float32)
        + bg_ref[...])
    proj = jnp.tanh(
        jnp.dot(h, wp_ref[...], preferred_element_type=jnp.float32)
        + bp_ref[...])
    part = jnp.sum(gate * proj, axis=0, keepdims=True)

    @pl.when(i == 0)
    def _():
        acc_ref[...] = part

    @pl.when(i > 0)
    def _():
        acc_ref[...] += part

    @pl.when(i == pl.num_programs(0) - 1)
    def _():
        cat = jnp.concatenate([acc_ref[...], lat_ref[...]], axis=1)
        hid = jax.nn.relu(
            jnp.dot(cat, fc1_ref[...], preferred_element_type=jnp.float32)
            + fc1b_ref[...])
        logits = jnp.dot(hid, fc2_ref[...],
                         preferred_element_type=jnp.float32) + fc2b_ref[...]
        out_ref[...] = jax.nn.softmax(logits, axis=-1)


def _tc_readout(h, wg, bg, wp, bp, lat, fc1w, fc1b, fc2w, fc2b):
    return pl.pallas_call(
        _readout_body,
        grid=(NB,),
        in_specs=[
            pl.BlockSpec((RB, D), lambda i: (i, 0)),
            pl.BlockSpec((D, D), lambda i: (0, 0)),
            pl.BlockSpec((1, D), lambda i: (0, 0)),
            pl.BlockSpec((D, D), lambda i: (0, 0)),
            pl.BlockSpec((1, D), lambda i: (0, 0)),
            pl.BlockSpec((1, D), lambda i: (0, 0)),
            pl.BlockSpec((2 * D, D), lambda i: (0, 0)),
            pl.BlockSpec((1, D), lambda i: (0, 0)),
            pl.BlockSpec((D, NF), lambda i: (0, 0)),
            pl.BlockSpec((1, NF), lambda i: (0, 0)),
        ],
        out_specs=pl.BlockSpec((1, NF), lambda i: (0, 0)),
        out_shape=jax.ShapeDtypeStruct((1, NF), jnp.float32),
        scratch_shapes=[pltpu.VMEM((1, D), jnp.float32)],
    )(h, wg, bg, wp, bp, lat, fc1w, fc1b, fc2w, fc2b)


def kernel(x, edge_index, edge_attr, latent_vector, Wm, bm, Wih, Whh, bih,
           bhh, gWm, gbm, gWih, gWhh, gbih, gbhh, Wg, bg, Wp, bp, fc1_W,
           fc1_b, fc2_W, fc2_b):
    src_r = edge_index[0].reshape(NW, NCH, CH)
    dst_r = edge_index[1].reshape(NW, NCH, CH)
    ea_ext = jnp.concatenate(
        [edge_attr,
         jnp.ones((E, 1), jnp.float32),
         jnp.zeros((E, EW - DE - 1), jnp.float32)], axis=1
    ).reshape(NW, NCH, CH, EW)
    zer_d = jnp.zeros((RPT, D), jnp.float32)
    zer_e = jnp.zeros((RPT, EW), jnp.float32)

    WmA = [Wm[0, :D], Wm[1, :D], gWm[0, :D], gWm[1, :D]]
    pad = jnp.zeros((EW - DE - 1, D), jnp.float32)
    Wext = [jnp.concatenate([Wm[0, D:], bm[0][None], pad], 0),
            jnp.concatenate([Wm[1, D:], bm[1][None], pad], 0),
            jnp.concatenate([gWm[0, D:], gbm[0][None], pad], 0),
            jnp.concatenate([gWm[1, D:], gbm[1][None], pad], 0)]
    WihL = [Wih[0], Wih[1], gWih[0], gWih[1]]
    WhhL = [Whh[0], Whh[1], gWhh[0], gWhh[1]]
    bihL = [bih[0][None], bih[1][None], gbih[0][None], gbih[1][None]]
    bhhL = [bhh[0][None], bhh[1][None], gbhh[0][None], gbhh[1][None]]

    eagg = _sc_eagg(dst_r, ea_ext, zer_e)

    h = x
    hm = _tc_matmul(x, WmA[0])
    for l in range(4):
        spm = _sc_spmm(src_r, dst_r, hm, zer_d)
        wnext = WmA[l + 1] if l < 3 else None
        h, hm = _tc_gru(h, spm, eagg, Wext[l], WihL[l], WhhL[l],
                        bihL[l], bhhL[l], wnext)

    return _tc_readout(h, Wg, bg[None], Wp, bp[None], latent_vector,
                       fc1_W, fc1_b[None], fc2_W, fc2_b[None])


# speedup vs baseline: 7.7255x; 1.0565x over previous
import functools

import jax
import jax.numpy as jnp
from jax import lax
from jax.experimental import pallas as pl
from jax.experimental.pallas import tpu as pltpu
from jax.experimental.pallas import tpu_sc as plsc

N = 10000
E = 320000
D = 128
DE = 16
NF = 64

NC = 2
NS = 16
NW = NC * NS
EPW = E // NW
CH = 125
NCH = EPW // CH
NPAD = 10112
RPT = NPAD // NS
EW = 32
CHE = 80
NCHE = EPW // CHE

def _sc_spmm_body(src_hbm, dst_hbm, hm_hbm, zer_hbm, out_hbm,
                  src_v, dst_v, gb0, gb1,
                  is0, is1, is2, is3, gs0, gs1, as0, as1, agg_sh):
    c = lax.axis_index("c")
    s = lax.axis_index("s")
    wid = s * NC + c
    pltpu.sync_copy(zer_hbm, agg_sh.at[pl.ds(s * RPT, RPT)])
    pltpu.sync_copy(dst_hbm.at[wid], dst_v)
    pltpu.sync_copy(src_hbm.at[wid, pl.ds(0, 3)], src_v.at[pl.ds(0, 3)])
    plsc.subcore_barrier()

    gbufs = [gb0, gb1]
    gsems = [gs0, gs1]
    asems = [as0, as1]
    isems = [is0, is1, is2, is3]

    pltpu.async_copy(hm_hbm.at[src_v.at[0]], gbufs[0], gsems[0])

    def rnd(r, carry):
        i0 = r * 4
        for b in range(4):
            ch = i0 + b
            b2 = b % 2
            pltpu.make_async_copy(hm_hbm.at[src_v.at[b]], gbufs[b2],
                                  gsems[b2]).wait()
            pltpu.async_copy(gbufs[b2], agg_sh.at[dst_v.at[ch]],
                             asems[b2], add=True)

            @pl.when(ch + 3 < NCH)
            def _():
                pltpu.async_copy(src_hbm.at[wid].at[ch + 3],
                                 src_v.at[(b + 3) % 4], isems[(b + 3) % 4])

            @pl.when(ch + 1 < NCH)
            def _():
                @pl.when(ch >= 1)
                def _():
                    pltpu.make_async_copy(
                        gbufs[1 - b2], agg_sh.at[dst_v.at[0]],
                        asems[1 - b2]).wait()

                @pl.when(ch + 1 >= 3)
                def _():
                    pltpu.make_async_copy(
                        src_hbm.at[wid].at[0], src_v.at[(b + 1) % 4],
                        isems[(b + 1) % 4]).wait()
                pltpu.async_copy(hm_hbm.at[src_v.at[(b + 1) % 4]],
                                 gbufs[1 - b2], gsems[1 - b2])
        return carry

    lax.fori_loop(0, NCH // 4, rnd, 0, unroll=False)
    for b2 in range(2):
        pltpu.make_async_copy(gbufs[b2], agg_sh.at[dst_v.at[0]],
                              asems[b2]).wait()
    plsc.subcore_barrier()
    pltpu.sync_copy(agg_sh.at[pl.ds(s * RPT, RPT)],
                    out_hbm.at[c].at[pl.ds(s * RPT, RPT)])


def _sc_eagg_body(dst_hbm, ea_hbm, zer_hbm, out_hbm,
                  dst_v, pbuf0, pbuf1, sem0, sem1, agg_sh):
    c = lax.axis_index("c")
    s = lax.axis_index("s")
    wid = s * NC + c
    pltpu.sync_copy(zer_hbm, agg_sh.at[pl.ds(s * RPT, RPT)])
    pltpu.sync_copy(dst_hbm.at[wid], dst_v)
    plsc.subcore_barrier()

    pltpu.async_copy(ea_hbm.at[wid].at[pl.ds(0, CHE)], pbuf0, sem0)
    npair = (NCHE - 1) // 2

    def pair(i, carry):
        a = 2 * i
        pltpu.async_copy(ea_hbm.at[wid].at[pl.ds((a + 1) * CHE, CHE)],
                         pbuf1, sem1)
        pltpu.make_async_copy(ea_hbm.at[wid].at[pl.ds(a * CHE, CHE)],
                              pbuf0, sem0).wait()
        pltpu.sync_copy(pbuf0, agg_sh.at[dst_v.at[a]], add=True)

        pltpu.async_copy(ea_hbm.at[wid].at[pl.ds((a + 2) * CHE, CHE)],
                         pbuf0, sem0)

        pltpu.make_async_copy(ea_hbm.at[wid].at[pl.ds((a + 1) * CHE, CHE)],
                              pbuf1, sem1).wait()
        pltpu.sync_copy(pbuf1, agg_sh.at[dst_v.at[a + 1]], add=True)
        return carry

    lax.fori_loop(0, npair, pair, 0, unroll=False)
    pltpu.make_async_copy(ea_hbm.at[wid].at[pl.ds((NCHE - 1) * CHE, CHE)],
                          pbuf0, sem0).wait()
    pltpu.sync_copy(pbuf0, agg_sh.at[dst_v.at[NCHE - 1]], add=True)
    plsc.subcore_barrier()
    pltpu.sync_copy(agg_sh.at[pl.ds(s * RPT, RPT)],
                    out_hbm.at[c].at[pl.ds(s * RPT, RPT)])


@functools.lru_cache(maxsize=None)
def _sc_kernels():
    mesh = plsc.VectorSubcoreMesh(core_axis_name="c", subcore_axis_name="s")
    spmm = pl.kernel(
        _sc_spmm_body,
        out_type=jax.ShapeDtypeStruct((NC, NPAD, D), jnp.float32),
        mesh=mesh,
        scratch_types=[
            pltpu.VMEM((4, CH), jnp.int32),
            pltpu.VMEM((NCH, CH), jnp.int32),
            pltpu.VMEM((CH, D), jnp.float32),
            pltpu.VMEM((CH, D), jnp.float32),
            pltpu.SemaphoreType.DMA,
            pltpu.SemaphoreType.DMA,
            pltpu.SemaphoreType.DMA,
            pltpu.SemaphoreType.DMA,
            pltpu.SemaphoreType.DMA,
            pltpu.SemaphoreType.DMA,
            pltpu.SemaphoreType.DMA,
            pltpu.SemaphoreType.DMA,
            pltpu.VMEM_SHARED((NPAD, D), jnp.float32),
        ],
    )
    eagg = pl.kernel(
        _sc_eagg_body,
        out_type=jax.ShapeDtypeStruct((NC, NPAD, EW), jnp.float32),
        mesh=mesh,
        scratch_types=[
            pltpu.VMEM((NCHE, CHE), jnp.int32),
            pltpu.VMEM((CHE, EW), jnp.float32),
            pltpu.VMEM((CHE, EW), jnp.float32),
            pltpu.SemaphoreType.DMA,
            pltpu.SemaphoreType.DMA,
            pltpu.VMEM_SHARED((NPAD, EW), jnp.float32),
        ],
    )
    return spmm, eagg


def _sc_spmm(src_r, dst_r, hm, zer_d):
    return _sc_kernels()[0](src_r, dst_r, hm, zer_d)


def _sc_eagg(dst_e, ea_ext, zer_e):
    return _sc_kernels()[1](dst_e, ea_ext, zer_e)


def _pad_body(x_ref, o_ref):
    x = x_ref[...]
    o_ref[...] = jnp.concatenate(
        [x,
         jnp.ones(x.shape[:-1] + (1,), jnp.float32),
         jnp.zeros(x.shape[:-1] + (EW - DE - 1,), jnp.float32)], axis=-1)


def _tc_pad(ea):
    return pl.pallas_call(
        _pad_body,
        grid=(NW,),
        in_specs=[pl.BlockSpec((1, EPW, DE), lambda i: (i, 0, 0))],
        out_specs=pl.BlockSpec((1, EPW, EW), lambda i: (i, 0, 0)),
        out_shape=jax.ShapeDtypeStruct((NW, EPW, EW), jnp.float32),
    )(ea)


RB = 1000
NB = N // RB


def _mm_body(x_ref, w_ref, o_ref):
    o_ref[...] = jnp.dot(x_ref[...], w_ref[...],
                         preferred_element_type=jnp.float32)


def _tc_matmul(x, w):
    m, k = x.shape
    _, n = w.shape
    return pl.pallas_call(
        _mm_body,
        grid=(m // RB,),
        in_specs=[
            pl.BlockSpec((RB, k), lambda i: (i, 0)),
            pl.BlockSpec((k, n), lambda i: (0, 0)),
        ],
        out_specs=pl.BlockSpec((RB, n), lambda i: (i, 0)),
        out_shape=jax.ShapeDtypeStruct((m, n), jnp.float32),
    )(x, w)


def _gru_body(with_next, h_ref, spm_ref, eagg_ref, wext_ref, wih_ref, whh_ref,
              bih_ref, bhh_ref, wnext_ref, hout_ref, hmout_ref):
    h = h_ref[...]
    agg = (spm_ref[0] + spm_ref[1]
           + jnp.dot(eagg_ref[0] + eagg_ref[1], wext_ref[...],
                     preferred_element_type=jnp.float32))
    gi = jnp.dot(agg, wih_ref[...], preferred_element_type=jnp.float32) \
        + bih_ref[...]
    gh = jnp.dot(h, whh_ref[...], preferred_element_type=jnp.float32) \
        + bhh_ref[...]
    r = jax.nn.sigmoid(gi[:, :D] + gh[:, :D])
    z = jax.nn.sigmoid(gi[:, D:2 * D] + gh[:, D:2 * D])
    n = jnp.tanh(gi[:, 2 * D:] + r * gh[:, 2 * D:])
    hn = (1.0 - z) * n + z * h
    hout_ref[...] = hn
    if with_next:
        hmout_ref[...] = jnp.dot(hn, wnext_ref[...],
                                 preferred_element_type=jnp.float32)


def _tc_gru(h, spm, eagg, wext, wih, whh, bih, bhh, wnext):
    with_next = wnext is not None
    if not with_next:
        wnext = jnp.zeros((D, D), jnp.float32)
    out_shapes = [jax.ShapeDtypeStruct((N, D), jnp.float32),
                  jax.ShapeDtypeStruct((N, D), jnp.float32)]
    outs = pl.pallas_call(
        functools.partial(_gru_body, with_next),
        grid=(NB,),
        in_specs=[
            pl.BlockSpec((RB, D), lambda i: (i, 0)),
            pl.BlockSpec((NC, RB, D), lambda i: (0, i, 0)),
            pl.BlockSpec((NC, RB, EW), lambda i: (0, i, 0)),
            pl.BlockSpec((EW, D), lambda i: (0, 0)),
            pl.BlockSpec((D, 3 * D), lambda i: (0, 0)),
            pl.BlockSpec((D, 3 * D), lambda i: (0, 0)),
            pl.BlockSpec((1, 3 * D), lambda i: (0, 0)),
            pl.BlockSpec((1, 3 * D), lambda i: (0, 0)),
            pl.BlockSpec((D, D), lambda i: (0, 0)),
        ],
        out_specs=[pl.BlockSpec((RB, D), lambda i: (i, 0)),
                   pl.BlockSpec((RB, D), lambda i: (i, 0))],
        out_shape=out_shapes,
    )(h, spm, eagg, wext, wih, whh, bih, bhh, wnext)
    return outs


def _readout_body(h_ref, wg_ref, bg_ref, wp_ref, bp_ref, lat_ref,
                  fc1_ref, fc1b_ref, fc2_ref, fc2b_ref, out_ref, acc_ref):
    i = pl.program_id(0)
    h = h_ref[...]
    gate = jax.nn.sigmoid(
        jnp.dot(h, wg_ref[...], preferred_element_type=jnp.float32)
        + bg_ref[...])
    proj = jnp.tanh(
        jnp.dot(h, wp_ref[...], preferred_element_type=jnp.float32)
        + bp_ref[...])
    part = jnp.sum(gate * proj, axis=0, keepdims=True)

    @pl.when(i == 0)
    def _():
        acc_ref[...] = part

    @pl.when(i > 0)
    def _():
        acc_ref[...] += part

    @pl.when(i == pl.num_programs(0) - 1)
    def _():
        cat = jnp.concatenate([acc_ref[...], lat_ref[...]], axis=1)
        hid = jax.nn.relu(
            jnp.dot(cat, fc1_ref[...], preferred_element_type=jnp.float32)
            + fc1b_ref[...])
        logits = jnp.dot(hid, fc2_ref[...],
                         preferred_element_type=jnp.float32) + fc2b_ref[...]
        out_ref[...] = jax.nn.softmax(logits, axis=-1)


def _tc_readout(h, wg, bg, wp, bp, lat, fc1w, fc1b, fc2w, fc2b):
    return pl.pallas_call(
        _readout_body,
        grid=(NB,),
        in_specs=[
            pl.BlockSpec((RB, D), lambda i: (i, 0)),
            pl.BlockSpec((D, D), lambda i: (0, 0)),
            pl.BlockSpec((1, D), lambda i: (0, 0)),
            pl.BlockSpec((D, D), lambda i: (0, 0)),
            pl.BlockSpec((1, D), lambda i: (0, 0)),
            pl.BlockSpec((1, D), lambda i: (0, 0)),
            pl.BlockSpec((2 * D, D), lambda i: (0, 0)),
            pl.BlockSpec((1, D), lambda i: (0, 0)),
            pl.BlockSpec((D, NF), lambda i: (0, 0)),
            pl.BlockSpec((1, NF), lambda i: (0, 0)),
        ],
        out_specs=pl.BlockSpec((1, NF), lambda i: (0, 0)),
        out_shape=jax.ShapeDtypeStruct((1, NF), jnp.float32),
        scratch_shapes=[pltpu.VMEM((1, D), jnp.float32)],
    )(h, wg, bg, wp, bp, lat, fc1w, fc1b, fc2w, fc2b)


def kernel(x, edge_index, edge_attr, latent_vector, Wm, bm, Wih, Whh, bih,
           bhh, gWm, gbm, gWih, gWhh, gbih, gbhh, Wg, bg, Wp, bp, fc1_W,
           fc1_b, fc2_W, fc2_b):
    src_r = edge_index[0].reshape(NW, NCH, CH)
    dst_r = edge_index[1].reshape(NW, NCH, CH)
    dst_e = edge_index[1].reshape(NW, NCHE, CHE)
    ea_w = edge_attr.reshape(NW, EPW, DE)
    zer_d = jnp.zeros((RPT, D), jnp.float32)
    zer_e = jnp.zeros((RPT, EW), jnp.float32)

    WmA = [Wm[0, :D], Wm[1, :D], gWm[0, :D], gWm[1, :D]]
    pad = jnp.zeros((EW - DE - 1, D), jnp.float32)
    Wext = [jnp.concatenate([Wm[0, D:], bm[0][None], pad], 0),
            jnp.concatenate([Wm[1, D:], bm[1][None], pad], 0),
            jnp.concatenate([gWm[0, D:], gbm[0][None], pad], 0),
            jnp.concatenate([gWm[1, D:], gbm[1][None], pad], 0)]
    WihL = [Wih[0], Wih[1], gWih[0], gWih[1]]
    WhhL = [Whh[0], Whh[1], gWhh[0], gWhh[1]]
    bihL = [bih[0][None], bih[1][None], gbih[0][None], gbih[1][None]]
    bhhL = [bhh[0][None], bhh[1][None], gbhh[0][None], gbhh[1][None]]

    ea_ext = jnp.concatenate(
        [ea_w, jnp.ones((NW, EPW, 1), jnp.float32),
         jnp.zeros((NW, EPW, EW - DE - 1), jnp.float32)], axis=2)
    eagg = _sc_eagg(dst_e, ea_ext, zer_e)

    h = x
    hm = _tc_matmul(x, WmA[0])
    for l in range(4):
        spm = _sc_spmm(src_r, dst_r, hm, zer_d)
        wnext = WmA[l + 1] if l < 3 else None
        h, hm = _tc_gru(h, spm, eagg, Wext[l], WihL[l], WhhL[l],
                        bihL[l], bhhL[l], wnext)

    return _tc_readout(h, Wg, bg[None], Wp, bp[None], latent_vector,
                       fc1_W, fc1_b[None], fc2_W, fc2_b[None])
